# baseline (speedup 1.0000x reference)
import functools

import jax
import jax.numpy as jnp
from jax import lax
from jax.experimental import pallas as pl
from jax.experimental.pallas import tpu as pltpu
from jax.experimental.pallas import tpu_sc as plsc

NC = 2
NS = 16
L = 16
B = 16384
K = 16
BPW = B // (NC * NS)
NBLK = BPW // L
SCALE = 0.5 / B


def _splfm_body(*refs):
  (sampleU, sampleI, sampleF, sampleR, alpha_arr,
   betaU, betaI, thetaU) = refs[:8]
  gU = refs[8:8 + K]
  gI = refs[8 + K:8 + 2 * K]
  (out_hbm, idxU_v, idxI_v, f_v, r_v, alpha_v,
   bu_v, bi_v, tu_v, gup_v, gip_v,
   lacc_v, out_v, allp_v, shared, sem) = refs[8 + 2 * K:]

  cid = lax.axis_index("c")
  sid = lax.axis_index("s")
  wid = sid * NC + cid
  base = wid * BPW

  pltpu.sync_copy(sampleU.at[pl.ds(base, BPW)], idxU_v)
  pltpu.sync_copy(sampleI.at[pl.ds(base, BPW)], idxI_v)
  pltpu.sync_copy(sampleF.at[pl.ds(base, BPW)], f_v)
  pltpu.sync_copy(sampleR.at[pl.ds(base, BPW)], r_v)
  pltpu.sync_copy(alpha_arr, alpha_v)

  cps = [
      pltpu.async_copy(betaU.at[idxU_v], bu_v, sem),
      pltpu.async_copy(betaI.at[idxI_v], bi_v, sem),
      pltpu.async_copy(thetaU.at[idxU_v], tu_v, sem),
  ]
  for k in range(K):
    dst = pl.ds(k * BPW, BPW)
    cps.append(pltpu.async_copy(gU[k].at[idxU_v], gup_v.at[dst], sem))
    cps.append(pltpu.async_copy(gI[k].at[idxI_v], gip_v.at[dst], sem))
  for cp in cps:
    cp.wait()

  av = alpha_v[...]

  def block_body(blk, acc):
    r0 = blk * L
    bu = bu_v[pl.ds(r0, L)]
    bi = bi_v[pl.ds(r0, L)]
    tu = tu_v[pl.ds(r0, L)]
    f = f_v[pl.ds(r0, L)]
    r = r_v[pl.ds(r0, L)]
    dot = jnp.zeros((L,), jnp.float32)
    for k in range(K):
      dot = dot + (gup_v[pl.ds(k * BPW + r0, L)] *
                   gip_v[pl.ds(k * BPW + r0, L)])
    e = av + bu + bi + tu * f + dot - r
    return acc + e * e

  loss_acc = lax.fori_loop(0, NBLK, block_body, jnp.zeros((L,), jnp.float32))

  lacc_v[...] = loss_acc
  pltpu.sync_copy(lacc_v, shared.at[sid])
  plsc.subcore_barrier()

  @pl.when(sid == 0)
  def _():
    pltpu.sync_copy(shared, allp_v)
    tot = jnp.zeros((L,), jnp.float32)
    for i in range(NS):
      tot = tot + allp_v[i, :]
    total = jnp.sum(tot) * SCALE
    out_v[...] = jnp.full((L,), total, jnp.float32)
    pltpu.sync_copy(out_v, out_hbm.at[pl.ds(cid * L, L)])


_splfm_call = functools.partial(
    pl.kernel,
    out_type=jax.ShapeDtypeStruct((NC * L,), jnp.float32),
    mesh=plsc.VectorSubcoreMesh(core_axis_name="c", subcore_axis_name="s"),
    compiler_params=pltpu.CompilerParams(
        needs_layout_passes=False, use_tc_tiling_on_sc=False),
    scratch_types=[
        pltpu.VMEM((BPW,), jnp.int32),
        pltpu.VMEM((BPW,), jnp.int32),
        pltpu.VMEM((BPW,), jnp.float32),
        pltpu.VMEM((BPW,), jnp.float32),
        pltpu.VMEM((L,), jnp.float32),
        pltpu.VMEM((BPW,), jnp.float32),
        pltpu.VMEM((BPW,), jnp.float32),
        pltpu.VMEM((BPW,), jnp.float32),
        pltpu.VMEM((K * BPW,), jnp.float32),
        pltpu.VMEM((K * BPW,), jnp.float32),
        pltpu.VMEM((L,), jnp.float32),
        pltpu.VMEM((L,), jnp.float32),
        pltpu.VMEM((NS, L), jnp.float32),
        pltpu.VMEM_SHARED((NS, L), jnp.float32),
        pltpu.SemaphoreType.DMA,
    ],
)(_splfm_body)


_N = 1000000


def _detile_body(gtu_ref, gti_ref, *rest):
  outs = rest[:2 * K]
  sem = rest[2 * K]
  cps = [
      pltpu.make_async_copy(ref.at[k], outs[t * K + k], sem)
      for t, ref in enumerate((gtu_ref, gti_ref))
      for k in range(K)
  ]
  for cp in cps:
    cp.start()
  for cp in cps:
    cp.wait()


_detile = pl.pallas_call(
    _detile_body,
    in_specs=[pl.BlockSpec(memory_space=pl.ANY)] * 2,
    out_specs=[pl.BlockSpec(memory_space=pl.ANY)] * (2 * K),
    out_shape=[jax.ShapeDtypeStruct((_N,), jnp.float32)] * (2 * K),
    scratch_shapes=[pltpu.SemaphoreType.DMA],
)


@jax.jit
def kernel(sampleU, sampleI, sampleF, sampleR, alpha,
           betaU, betaI, thetaU, gammaU, gammaI):
  alpha_arr = jnp.full((L,), alpha, dtype=jnp.float32)
  planes = _detile(gammaU.T, gammaI.T)
  gU = planes[:K]
  gI = planes[K:]
  out = _splfm_call(sampleU, sampleI, sampleF, sampleR, alpha_arr,
                    betaU, betaI, thetaU, *gU, *gI)
  return out[0] + out[L]


# speedup vs baseline: 30.6546x; 30.6546x over previous
import functools

import jax
import jax.numpy as jnp
from jax import lax
from jax.experimental import pallas as pl
from jax.experimental.pallas import tpu as pltpu
from jax.experimental.pallas import tpu_sc as plsc

NC = 2
NS = 16
L = 16
B = 16384
K = 16
BPW = B // (NC * NS)
NBLK = BPW // L
SCALE = 0.5 / B


def _splfm_body(*refs):
  (sampleU, sampleI, sampleF, sampleR, alpha_arr,
   betaU, betaI, thetaU) = refs[:8]
  gU = refs[8:8 + K]
  gI = refs[8 + K:8 + 2 * K]
  (out_hbm, idxU_v, idxI_v, f_v, r_v, alpha_v,
   bu_v, bi_v, tu_v, gup_v, gip_v,
   lacc_v, out_v, allp_v, shared, sem) = refs[8 + 2 * K:]

  cid = lax.axis_index("c")
  sid = lax.axis_index("s")
  wid = sid * NC + cid
  base = wid * BPW

  pltpu.sync_copy(sampleU.at[pl.ds(base, BPW)], idxU_v)
  pltpu.sync_copy(sampleI.at[pl.ds(base, BPW)], idxI_v)
  pltpu.sync_copy(sampleF.at[pl.ds(base, BPW)], f_v)
  pltpu.sync_copy(sampleR.at[pl.ds(base, BPW)], r_v)
  pltpu.sync_copy(alpha_arr, alpha_v)

  cps = [
      pltpu.async_copy(betaU.at[idxU_v], bu_v, sem),
      pltpu.async_copy(betaI.at[idxI_v], bi_v, sem),
      pltpu.async_copy(thetaU.at[idxU_v], tu_v, sem),
  ]
  for k in range(K):
    dst = pl.ds(k * BPW, BPW)
    cps.append(pltpu.async_copy(gU[k].at[idxU_v], gup_v.at[dst], sem))
    cps.append(pltpu.async_copy(gI[k].at[idxI_v], gip_v.at[dst], sem))
  for cp in cps:
    cp.wait()

  av = alpha_v[...]

  def block_body(blk, acc):
    r0 = blk * L
    bu = bu_v[pl.ds(r0, L)]
    bi = bi_v[pl.ds(r0, L)]
    tu = tu_v[pl.ds(r0, L)]
    f = f_v[pl.ds(r0, L)]
    r = r_v[pl.ds(r0, L)]
    dot = jnp.zeros((L,), jnp.float32)
    for k in range(K):
      dot = dot + (gup_v[pl.ds(k * BPW + r0, L)] *
                   gip_v[pl.ds(k * BPW + r0, L)])
    e = av + bu + bi + tu * f + dot - r
    return acc + e * e

  loss_acc = lax.fori_loop(0, NBLK, block_body, jnp.zeros((L,), jnp.float32))

  lacc_v[...] = loss_acc
  pltpu.sync_copy(lacc_v, shared.at[sid])
  plsc.subcore_barrier()

  @pl.when(sid == 0)
  def _():
    pltpu.sync_copy(shared, allp_v)
    tot = jnp.zeros((L,), jnp.float32)
    for i in range(NS):
      tot = tot + allp_v[i, :]
    total = jnp.sum(tot) * SCALE
    out_v[...] = jnp.full((L,), total, jnp.float32)
    pltpu.sync_copy(out_v, out_hbm.at[pl.ds(cid * L, L)])


_splfm_call = functools.partial(
    pl.kernel,
    out_type=jax.ShapeDtypeStruct((NC * L,), jnp.float32),
    mesh=plsc.VectorSubcoreMesh(core_axis_name="c", subcore_axis_name="s"),
    compiler_params=pltpu.CompilerParams(
        needs_layout_passes=False, use_tc_tiling_on_sc=False),
    scratch_types=[
        pltpu.VMEM((BPW,), jnp.int32),
        pltpu.VMEM((BPW,), jnp.int32),
        pltpu.VMEM((BPW,), jnp.float32),
        pltpu.VMEM((BPW,), jnp.float32),
        pltpu.VMEM((L,), jnp.float32),
        pltpu.VMEM((BPW,), jnp.float32),
        pltpu.VMEM((BPW,), jnp.float32),
        pltpu.VMEM((BPW,), jnp.float32),
        pltpu.VMEM((K * BPW,), jnp.float32),
        pltpu.VMEM((K * BPW,), jnp.float32),
        pltpu.VMEM((L,), jnp.float32),
        pltpu.VMEM((L,), jnp.float32),
        pltpu.VMEM((NS, L), jnp.float32),
        pltpu.VMEM_SHARED((NS, L), jnp.float32),
        pltpu.SemaphoreType.DMA,
    ],
)(_splfm_body)


_N = 1000000
_CHUNK = 65536
_NCHUNK = -(-_N // _CHUNK)


def _detile_body(gtu_ref, gti_ref, *out_refs):
  for k in range(K):
    out_refs[k][...] = gtu_ref[k, :]
    out_refs[K + k][...] = gti_ref[k, :]


_detile = pl.pallas_call(
    _detile_body,
    grid=(_NCHUNK,),
    in_specs=[pl.BlockSpec((K, _CHUNK), lambda c: (0, c))] * 2,
    out_specs=[pl.BlockSpec((_CHUNK,), lambda c: (c,))] * (2 * K),
    out_shape=[jax.ShapeDtypeStruct((_N,), jnp.float32)] * (2 * K),
)


@jax.jit
def kernel(sampleU, sampleI, sampleF, sampleR, alpha,
           betaU, betaI, thetaU, gammaU, gammaI):
  alpha_arr = jnp.full((L,), alpha, dtype=jnp.float32)
  planes = _detile(gammaU.T, gammaI.T)
  gU = planes[:K]
  gI = planes[K:]
  out = _splfm_call(sampleU, sampleI, sampleF, sampleR, alpha_arr,
                    betaU, betaI, thetaU, *gU, *gI)
  return out[0] + out[L]
